# Initial kernel scaffold; baseline (speedup 1.0000x reference)
import jax
import jax.numpy as jnp
from jax.experimental import pallas as pl


def kernel(x, gate_w, gate_b, experts_w, experts_b, noise):
    raise NotImplementedError("write your pallas kernel here")


# speedup vs baseline: 1.2326x; 1.2326x over previous
import functools

import jax
import jax.numpy as jnp
from jax import lax
from jax.experimental import pallas as pl
from jax.experimental.pallas import tpu as pltpu
from jax.experimental.pallas import tpu_sc as plsc

TOK = 8192
D_IN = 2048
D_OUT = 2048
E = 8
K = 2
NOISE_STD = 0.1

M = 256
NASSIGN = TOK * K
NPAD = NASSIGN + E * M
NBLK = NPAD // M

NC = 2
NS = 16
NW = NC * NS
RW = NPAD // NW
TW = TOK // NW
GC = 16
CC = 16

_LANES = 128


def _gating_body(x_ref, gw_ref, nz_ref, gb_ref, pv_ref, iv_ref, loss_ref,
                 acc_ref):
    i = pl.program_id(0)
    logits = jnp.dot(x_ref[...], gw_ref[...],
                     preferred_element_type=jnp.float32)
    logits = logits + gb_ref[...] + nz_ref[...] * NOISE_STD
    lane = lax.broadcasted_iota(jnp.int32, logits.shape, 1)
    valid = lane < E
    logits = jnp.where(valid, logits, jnp.float32(-1e30))
    m = jnp.max(logits, axis=1, keepdims=True)
    e = jnp.where(valid, jnp.exp(logits - m), 0.0)
    probs = e / jnp.sum(e, axis=1, keepdims=True)

    @pl.when(i == 0)
    def _():
        acc_ref[...] = jnp.zeros_like(acc_ref)

    acc_ref[...] += jnp.sum(probs, axis=0, keepdims=True)

    v1 = jnp.max(probs, axis=1, keepdims=True)
    i1 = jnp.min(jnp.where(probs == v1, lane, 999), axis=1, keepdims=True)
    probs2 = jnp.where(lane == i1, -1.0, probs)
    v2 = jnp.max(probs2, axis=1, keepdims=True)
    i2 = jnp.min(jnp.where(probs2 == v2, lane, 999), axis=1, keepdims=True)
    pv_ref[...] = jnp.concatenate([v1, v2], axis=1)
    iv_ref[...] = jnp.concatenate([i1, i2], axis=1)

    @pl.when(i == pl.num_programs(0) - 1)
    def _():
        mean = acc_ref[...] * (1.0 / TOK)
        lane2 = lax.broadcasted_iota(jnp.int32, mean.shape, 1)
        dev = jnp.where(lane2 < E, mean - 1.0 / E, 0.0)
        loss_ref[...] = jnp.sum(dev * dev).reshape(1, 1)


def _gating(x, gwp, noise_p, gbp):
    bt = 1024
    grid = TOK // bt
    return pl.pallas_call(
        _gating_body,
        grid=(grid,),
        in_specs=[
            pl.BlockSpec((bt, D_IN), lambda i: (i, 0)),
            pl.BlockSpec((D_IN, _LANES), lambda i: (0, 0)),
            pl.BlockSpec((bt, _LANES), lambda i: (i, 0)),
            pl.BlockSpec((1, _LANES), lambda i: (0, 0)),
        ],
        out_specs=[
            pl.BlockSpec((bt, K), lambda i: (i, 0)),
            pl.BlockSpec((bt, K), lambda i: (i, 0)),
            pl.BlockSpec((1, 1), lambda i: (0, 0)),
        ],
        out_shape=[
            jax.ShapeDtypeStruct((TOK, K), jnp.float32),
            jax.ShapeDtypeStruct((TOK, K), jnp.int32),
            jax.ShapeDtypeStruct((1, 1), jnp.float32),
        ],
        scratch_shapes=[pltpu.VMEM((1, _LANES), jnp.float32)],
    )(x, gwp, noise_p, gbp)


def _gather_body(x_hbm, idx_hbm, out_hbm, idx_v, rows_v, sem):
    wid = lax.axis_index("s") * NC + lax.axis_index("c")
    base = wid * RW

    def chunk(g, carry):
        off = base + g * GC
        pltpu.sync_copy(idx_hbm.at[pl.ds(off, GC)], idx_v)
        pltpu.async_copy(x_hbm.at[idx_v], rows_v, sem).wait()
        pltpu.sync_copy(rows_v, out_hbm.at[pl.ds(off, GC), :])
        return carry

    lax.fori_loop(0, RW // GC, chunk, 0)


def _sc_gather(x, src_tok):
    k = pl.kernel(
        _gather_body,
        mesh=plsc.VectorSubcoreMesh(core_axis_name="c", subcore_axis_name="s"),
        out_type=jax.ShapeDtypeStruct((NPAD, D_IN), jnp.float32),
        scratch_types=[
            pltpu.VMEM((GC,), jnp.int32),
            pltpu.VMEM((GC, D_IN), jnp.float32),
            pltpu.SemaphoreType.DMA,
        ],
    )
    return k(x, src_tok)


def _gmm_body(be_ref, xg_ref, pg_ref, wt_ref, eb_ref, out_ref):
    xs = xg_ref[...] * pg_ref[...]
    acc = jnp.dot(xs.astype(jnp.bfloat16), wt_ref[0],
                  preferred_element_type=jnp.float32)
    out_ref[...] = acc + pg_ref[...] * eb_ref[0]


def _gmm(block_expert, xg, pg, wt, eb):
    grid_spec = pltpu.PrefetchScalarGridSpec(
        num_scalar_prefetch=1,
        grid=(NBLK,),
        in_specs=[
            pl.BlockSpec((M, D_IN), lambda i, be: (i, 0)),
            pl.BlockSpec((M, 1), lambda i, be: (i, 0)),
            pl.BlockSpec((1, D_IN, D_OUT), lambda i, be: (be[i], 0, 0)),
            pl.BlockSpec((1, 1, D_OUT), lambda i, be: (be[i], 0, 0)),
        ],
        out_specs=pl.BlockSpec((M, D_OUT), lambda i, be: (i, 0)),
    )
    return pl.pallas_call(
        _gmm_body,
        grid_spec=grid_spec,
        out_shape=jax.ShapeDtypeStruct((NPAD, D_OUT), jnp.float32),
    )(block_expert, xg, pg, wt, eb)


def _combine_body(yg_hbm, d0_hbm, d1_hbm, out_hbm, i0_v, i1_v, a_v, b_v,
                  sem0, sem1):
    wid = lax.axis_index("s") * NC + lax.axis_index("c")
    base = wid * TW

    def chunk(g, carry):
        off = base + g * CC
        pltpu.sync_copy(d0_hbm.at[pl.ds(off, CC)], i0_v)
        pltpu.sync_copy(d1_hbm.at[pl.ds(off, CC)], i1_v)
        cp0 = pltpu.async_copy(yg_hbm.at[i0_v], a_v, sem0)
        cp1 = pltpu.async_copy(yg_hbm.at[i1_v], b_v, sem1)
        cp0.wait()
        cp1.wait()

        def row(i, c2):
            def vec8(j, c3):
                for u in range(8):
                    sl = pl.ds((j * 8 + u) * 16, 16)
                    a_v[i, sl] = a_v[i, sl] + b_v[i, sl]
                return c3
            return lax.fori_loop(0, D_OUT // 128, vec8, c2)

        lax.fori_loop(0, CC, row, 0)
        pltpu.sync_copy(a_v, out_hbm.at[pl.ds(off, CC), :])
        return carry

    lax.fori_loop(0, TW // CC, chunk, 0)


def _sc_combine(yg, d0, d1):
    k = pl.kernel(
        _combine_body,
        mesh=plsc.VectorSubcoreMesh(core_axis_name="c", subcore_axis_name="s"),
        out_type=jax.ShapeDtypeStruct((TOK, D_OUT), jnp.float32),
        scratch_types=[
            pltpu.VMEM((CC,), jnp.int32),
            pltpu.VMEM((CC,), jnp.int32),
            pltpu.VMEM((CC, D_OUT), jnp.float32),
            pltpu.VMEM((CC, D_OUT), jnp.float32),
            pltpu.SemaphoreType.DMA,
            pltpu.SemaphoreType.DMA,
        ],
    )
    return k(yg, d0, d1)


def kernel(x, gate_w, gate_b, experts_w, experts_b, noise):
    gwp = jnp.zeros((_LANES, D_IN), jnp.float32).at[:E].set(gate_w).T
    gbp = jnp.zeros((1, _LANES), jnp.float32).at[0, :E].set(gate_b)
    noise_p = jnp.zeros((TOK, _LANES), jnp.float32).at[:, :E].set(noise)
    wt = (experts_w.reshape(E, D_OUT, D_IN)
          .transpose(0, 2, 1).astype(jnp.bfloat16))
    eb = experts_b.reshape(E, 1, D_OUT)

    pv, iv, loss = _gating(x, gwp, noise_p, gbp)

    e_flat = iv.reshape(-1)
    p_flat = pv.reshape(-1)
    oh = (e_flat[:, None] == jnp.arange(E, dtype=jnp.int32)[None, :])
    csum = jnp.cumsum(oh.astype(jnp.int32), axis=0)
    rank = jnp.take_along_axis(csum, e_flat[:, None], axis=1)[:, 0] - 1
    counts = csum[-1]
    padded = ((counts + M - 1) // M) * M
    pcum = jnp.cumsum(padded)
    poff = jnp.concatenate([jnp.zeros((1,), pcum.dtype), pcum])[:E]
    dest = (poff[e_flat] + rank).astype(jnp.int32)
    arange_a = jnp.arange(NASSIGN, dtype=jnp.int32)
    src_tok = jnp.zeros((NPAD,), jnp.int32).at[dest].set(arange_a // K)
    pg = jnp.zeros((NPAD,), jnp.float32).at[dest].set(p_flat)
    bstart = jnp.arange(NBLK, dtype=jnp.int32) * M
    block_expert = jnp.minimum(
        jnp.sum((bstart[:, None] >= pcum[None, :]).astype(jnp.int32), axis=1),
        E - 1).astype(jnp.int32)
    d0 = dest[0::K]
    d1 = dest[1::K]

    xg = _sc_gather(x, src_tok)

    yg = _gmm(block_expert, xg, pg.reshape(NPAD, 1), wt, eb)

    out = _sc_combine(yg, d0, d1)

    return (out, loss[0, 0])
